# baseline (speedup 1.0000x reference)
import jax
import jax.numpy as jnp
from jax import lax
from jax.experimental import pallas as pl
from jax.experimental.pallas import tpu as pltpu
from jax.experimental.pallas import tpu_sc as plsc

V = 1000000
D = 64
H = 128
B = 4096
L = 50
N = B * L

NC, NS = 2, 16
CHUNK = 128
TOK_PER_SUB = N // NS
N_CHUNKS = TOK_PER_SUB // CHUNK


def _gather_body(idx1, idx2, table, g1, g2, idx_v, rows_v, sem):
    c = lax.axis_index("c")
    s = lax.axis_index("s")
    base = s * TOK_PER_SUB

    def run(idx_hbm, g_hbm):
        def body(i, carry):
            off = base + i * CHUNK
            pltpu.sync_copy(idx_hbm.at[pl.ds(off, CHUNK)], idx_v)
            pltpu.async_copy(table.at[idx_v], rows_v, sem).wait()
            pltpu.sync_copy(rows_v, g_hbm.at[pl.ds(off, CHUNK)])
            return carry
        lax.fori_loop(0, N_CHUNKS, body, 0)

    @pl.when(c == 0)
    def _():
        run(idx1, g1)

    @pl.when(c == 1)
    def _():
        run(idx2, g2)


_gather = pl.kernel(
    _gather_body,
    out_type=[
        jax.ShapeDtypeStruct((N, D), jnp.float32),
        jax.ShapeDtypeStruct((N, D), jnp.float32),
    ],
    mesh=plsc.VectorSubcoreMesh(core_axis_name="c", subcore_axis_name="s"),
    scratch_types=[
        pltpu.VMEM((CHUNK,), jnp.int32),
        pltpu.VMEM((CHUNK, D), jnp.float32),
        pltpu.SemaphoreType.DMA,
    ],
    compiler_params=pltpu.CompilerParams(use_tc_tiling_on_sc=False),
)


BM = 1024


def _mm_body(x_ref, w_ref, b_ref, o_ref):
    o_ref[...] = lax.dot_general(
        x_ref[...], w_ref[...],
        dimension_numbers=(((1,), (1,)), ((), ())),
        preferred_element_type=jnp.float32,
    ) + b_ref[...]


_mm = pl.pallas_call(
    _mm_body,
    grid=(N // BM,),
    in_specs=[
        pl.BlockSpec((BM, D), lambda i: (i, 0)),
        pl.BlockSpec((H, D), lambda i: (0, 0)),
        pl.BlockSpec((1, H), lambda i: (0, 0)),
    ],
    out_specs=pl.BlockSpec((BM, H), lambda i: (i, 0)),
    out_shape=jax.ShapeDtypeStruct((N, H), jnp.float32),
)


def kernel(sent1, sent2, emb_table, W, b):
    idx1 = sent1.reshape(-1).astype(jnp.int32)
    idx2 = sent2.reshape(-1).astype(jnp.int32)
    g1, g2 = _gather(idx1, idx2, emb_table)
    b2 = b.reshape(1, H)
    o1 = _mm(g1, W, b2)
    o2 = _mm(g2, W, b2)
    return o1.reshape(B, L, H), o2.reshape(B, L, H)


# speedup vs baseline: 1.2685x; 1.2685x over previous
import jax
import jax.numpy as jnp
from jax import lax
from jax.experimental import pallas as pl
from jax.experimental.pallas import tpu as pltpu
from jax.experimental.pallas import tpu_sc as plsc

V = 1000000
D = 64
H = 128
B = 4096
L = 50
N = B * L

NC, NS = 2, 16
CHUNK = 256
TOK_PER_SUB = N // NS
N_CHUNKS = TOK_PER_SUB // CHUNK


BMP = 4000


def _proj_body(t_ref, w_ref, b_ref, p_ref):
    p_ref[...] = lax.dot_general(
        t_ref[...], w_ref[...],
        dimension_numbers=(((1,), (1,)), ((), ())),
        preferred_element_type=jnp.float32,
    ) + b_ref[...]


_project = pl.pallas_call(
    _proj_body,
    grid=(V // BMP,),
    in_specs=[
        pl.BlockSpec((BMP, D), lambda i: (i, 0)),
        pl.BlockSpec((H, D), lambda i: (0, 0)),
        pl.BlockSpec((1, H), lambda i: (0, 0)),
    ],
    out_specs=pl.BlockSpec((BMP, H), lambda i: (i, 0)),
    out_shape=jax.ShapeDtypeStruct((V, H), jnp.float32),
)


def _gather_body(idx1, idx2, table, g1, g2, idx_v, rows_v, sem):
    c = lax.axis_index("c")
    s = lax.axis_index("s")
    base = s * TOK_PER_SUB

    def run(idx_hbm, g_hbm):
        def body(i, carry):
            off = base + i * CHUNK
            pltpu.sync_copy(idx_hbm.at[pl.ds(off, CHUNK)], idx_v)
            pltpu.async_copy(table.at[idx_v], rows_v, sem).wait()
            pltpu.sync_copy(rows_v, g_hbm.at[pl.ds(off, CHUNK)])
            return carry
        lax.fori_loop(0, N_CHUNKS, body, 0)

    @pl.when(c == 0)
    def _():
        run(idx1, g1)

    @pl.when(c == 1)
    def _():
        run(idx2, g2)


_gather = pl.kernel(
    _gather_body,
    out_type=[
        jax.ShapeDtypeStruct((N, H), jnp.float32),
        jax.ShapeDtypeStruct((N, H), jnp.float32),
    ],
    mesh=plsc.VectorSubcoreMesh(core_axis_name="c", subcore_axis_name="s"),
    scratch_types=[
        pltpu.VMEM((CHUNK,), jnp.int32),
        pltpu.VMEM((CHUNK, H), jnp.float32),
        pltpu.SemaphoreType.DMA,
    ],
    compiler_params=pltpu.CompilerParams(use_tc_tiling_on_sc=False),
)


def kernel(sent1, sent2, emb_table, W, b):
    idx1 = sent1.reshape(-1).astype(jnp.int32)
    idx2 = sent2.reshape(-1).astype(jnp.int32)
    p = _project(emb_table, W, b.reshape(1, H))
    g1, g2 = _gather(idx1, idx2, p)
    return g1.reshape(B, L, H), g2.reshape(B, L, H)


# speedup vs baseline: 1.3314x; 1.0496x over previous
import jax
import jax.numpy as jnp
from jax import lax
from jax.experimental import pallas as pl
from jax.experimental.pallas import tpu as pltpu
from jax.experimental.pallas import tpu_sc as plsc

V = 1000000
D = 64
H = 128
B = 4096
L = 50
N = B * L

NC, NS = 2, 16
NW = NC * NS
TOKW = N // NW
CHUNK = 256
NCH = TOKW // CHUNK


BMP = 4000


def _proj_body(t_ref, w_ref, b_ref, p_ref):
    p_ref[...] = lax.dot_general(
        t_ref[...], w_ref[...],
        dimension_numbers=(((1,), (1,)), ((), ())),
        preferred_element_type=jnp.float32,
    ) + b_ref[...]


_project = pl.pallas_call(
    _proj_body,
    grid=(V // BMP,),
    in_specs=[
        pl.BlockSpec((BMP, D), lambda i: (i, 0)),
        pl.BlockSpec((H, D), lambda i: (0, 0)),
        pl.BlockSpec((1, H), lambda i: (0, 0)),
    ],
    out_specs=pl.BlockSpec((BMP, H), lambda i: (i, 0)),
    out_shape=jax.ShapeDtypeStruct((V, H), jnp.float32),
)


def _gather_body(idx_hbm, p_hbm, g_hbm, idx_v, rows_v, sem_o):
    c = lax.axis_index("c")
    s = lax.axis_index("s")
    base = (c * NS + s) * TOKW

    pltpu.sync_copy(idx_hbm.at[pl.ds(base, TOKW)], idx_v)

    def it(i, carry):
        par = lax.rem(i, 2)
        off = base + i * CHUNK

        @pl.when(i >= 2)
        def _():
            pltpu.make_async_copy(
                rows_v.at[par], g_hbm.at[pl.ds(off, CHUNK)], sem_o.at[par]
            ).wait()

        pltpu.async_copy(
            p_hbm.at[idx_v.at[pl.ds(i * CHUNK, CHUNK)]], rows_v.at[par], sem_o.at[2]
        ).wait()
        pltpu.async_copy(rows_v.at[par], g_hbm.at[pl.ds(off, CHUNK)], sem_o.at[par])
        return carry

    lax.fori_loop(0, NCH, it, 0)

    for i in (NCH - 2, NCH - 1):
        par = i % 2
        pltpu.make_async_copy(
            rows_v.at[par], g_hbm.at[pl.ds(base + i * CHUNK, CHUNK)], sem_o.at[par]
        ).wait()


_gather = pl.kernel(
    _gather_body,
    out_type=jax.ShapeDtypeStruct((N, H), jnp.float32),
    mesh=plsc.VectorSubcoreMesh(core_axis_name="c", subcore_axis_name="s"),
    scratch_types=[
        pltpu.VMEM((TOKW,), jnp.int32),
        pltpu.VMEM((2, CHUNK, H), jnp.float32),
        pltpu.SemaphoreType.DMA((3,)),
    ],
    compiler_params=pltpu.CompilerParams(use_tc_tiling_on_sc=False),
)


RB = 32


def _repack_body(x_ref, o_ref):
    for j in range(RB):
        o_ref[j] = x_ref[pl.ds(j * L, L)]


_repack = pl.pallas_call(
    _repack_body,
    grid=(B // RB,),
    in_specs=[pl.BlockSpec((RB * L, H), lambda i: (i, 0))],
    out_specs=pl.BlockSpec((RB, L, H), lambda i: (i, 0, 0)),
    out_shape=jax.ShapeDtypeStruct((B, L, H), jnp.float32),
)


def kernel(sent1, sent2, emb_table, W, b):
    idx1 = sent1.reshape(-1).astype(jnp.int32)
    idx2 = sent2.reshape(-1).astype(jnp.int32)
    p = _project(emb_table, W, b.reshape(1, H))
    g1 = _gather(idx1, p)
    g2 = _gather(idx2, p)
    return _repack(g1), _repack(g2)


# speedup vs baseline: 1.5555x; 1.1683x over previous
import jax
import jax.numpy as jnp
from jax import lax
from jax.experimental import pallas as pl
from jax.experimental.pallas import tpu as pltpu
from jax.experimental.pallas import tpu_sc as plsc

V = 1000000
D = 64
H = 128
B = 4096
L = 50
N = B * L

NC, NS = 2, 16
NW = NC * NS
TOKW = N // NW
CHUNK = 256
NCH = TOKW // CHUNK


BM3 = 1000
BMP = BM3 * 8


def _proj_body(t_ref, w_ref, b_ref, p_ref):
    t = t_ref[...].reshape(BMP, D)
    p_ref[...] = lax.dot_general(
        t.astype(jnp.bfloat16), w_ref[...].astype(jnp.bfloat16),
        dimension_numbers=(((1,), (1,)), ((), ())),
        preferred_element_type=jnp.float32,
    ) + b_ref[...]


_project = pl.pallas_call(
    _proj_body,
    grid=(V // BMP,),
    in_specs=[
        pl.BlockSpec((BM3, 8, D), lambda i: (i, 0, 0)),
        pl.BlockSpec((H, D), lambda i: (0, 0)),
        pl.BlockSpec((1, H), lambda i: (0, 0)),
    ],
    out_specs=pl.BlockSpec((BMP, H), lambda i: (i, 0)),
    out_shape=jax.ShapeDtypeStruct((V, H), jnp.float32),
)


def _gather_body(idx_hbm, p_hbm, g_hbm, idx_v, rows_v, sem_o):
    c = lax.axis_index("c")
    s = lax.axis_index("s")
    base = (c * NS + s) * TOKW

    pltpu.sync_copy(idx_hbm.at[pl.ds(base, TOKW)], idx_v)

    def it(i, carry):
        par = lax.rem(i, 2)
        off = base + i * CHUNK

        @pl.when(i >= 2)
        def _():
            pltpu.make_async_copy(
                rows_v.at[par], g_hbm.at[pl.ds(off, CHUNK)], sem_o.at[par]
            ).wait()

        pltpu.async_copy(
            p_hbm.at[idx_v.at[pl.ds(i * CHUNK, CHUNK)]], rows_v.at[par], sem_o.at[2]
        ).wait()
        pltpu.async_copy(rows_v.at[par], g_hbm.at[pl.ds(off, CHUNK)], sem_o.at[par])
        return carry

    lax.fori_loop(0, NCH, it, 0)

    for i in (NCH - 2, NCH - 1):
        par = i % 2
        pltpu.make_async_copy(
            rows_v.at[par], g_hbm.at[pl.ds(base + i * CHUNK, CHUNK)], sem_o.at[par]
        ).wait()


_gather = pl.kernel(
    _gather_body,
    out_type=jax.ShapeDtypeStruct((N, H), jnp.float32),
    mesh=plsc.VectorSubcoreMesh(core_axis_name="c", subcore_axis_name="s"),
    scratch_types=[
        pltpu.VMEM((TOKW,), jnp.int32),
        pltpu.VMEM((2, CHUNK, H), jnp.float32),
        pltpu.SemaphoreType.DMA((3,)),
    ],
    compiler_params=pltpu.CompilerParams(use_tc_tiling_on_sc=False),
)


RB = 32


def _repack_body(x_ref, o_ref):
    for j in range(RB):
        o_ref[j] = x_ref[pl.ds(j * L, L)]


_repack = pl.pallas_call(
    _repack_body,
    grid=(B // RB,),
    in_specs=[pl.BlockSpec((RB * L, H), lambda i: (i, 0))],
    out_specs=pl.BlockSpec((RB, L, H), lambda i: (i, 0, 0)),
    out_shape=jax.ShapeDtypeStruct((B, L, H), jnp.float32),
)


def kernel(sent1, sent2, emb_table, W, b):
    idx1 = sent1.reshape(-1).astype(jnp.int32)
    idx2 = sent2.reshape(-1).astype(jnp.int32)
    p = _project(emb_table.reshape(V // 8, 8, D), W, b.reshape(1, H))
    g1 = _gather(idx1, p)
    g2 = _gather(idx2, p)
    return _repack(g1), _repack(g2)


# speedup vs baseline: 1.9235x; 1.2366x over previous
import jax
import jax.numpy as jnp
from jax import lax
from jax.experimental import pallas as pl
from jax.experimental.pallas import tpu as pltpu
from jax.experimental.pallas import tpu_sc as plsc

V = 1000000
D = 64
H = 128
B = 4096
L = 50

NC, NS = 2, 16
CB = 4
NB_W = B // NS
NITER = NB_W // CB


BM3 = 1000
BMP = BM3 * 8


def _proj_body(t_ref, w_ref, b_ref, p_ref):
    t = t_ref[...].reshape(BMP, D)
    p_ref[...] = lax.dot_general(
        t.astype(jnp.bfloat16), w_ref[...].astype(jnp.bfloat16),
        dimension_numbers=(((1,), (1,)), ((), ())),
        preferred_element_type=jnp.float32,
    ) + b_ref[...]


_project = pl.pallas_call(
    _proj_body,
    grid=(V // BMP,),
    in_specs=[
        pl.BlockSpec((BM3, 8, D), lambda i: (i, 0, 0)),
        pl.BlockSpec((H, D), lambda i: (0, 0)),
        pl.BlockSpec((1, H), lambda i: (0, 0)),
    ],
    out_specs=pl.BlockSpec((BMP, H), lambda i: (i, 0)),
    out_shape=jax.ShapeDtypeStruct((V, H), jnp.float32),
)


def _gather_body(idxp1, idxp2, p_hbm, g1, g2, idx_v, v3, sem_g, sem_w):
    c = lax.axis_index("c")
    s = lax.axis_index("s")
    b_base = s * NB_W

    def run(idxp, g_hbm):
        pltpu.sync_copy(idxp.at[pl.ds(b_base, NB_W)], idx_v)

        def it(j, carry):
            par = lax.rem(j, 2)
            bb = b_base + j * CB

            @pl.when(j >= 2)
            def _():
                pltpu.make_async_copy(
                    v3.at[par], g_hbm.at[pl.ds(bb - 2 * CB, CB)], sem_w.at[par]
                ).wait()

            cps = [
                pltpu.async_copy(
                    p_hbm.at[idx_v.at[j * CB + k, pl.ds(0, L)]],
                    v3.at[par, k],
                    sem_g,
                )
                for k in range(CB)
            ]
            for cp in cps:
                cp.wait()
            pltpu.async_copy(v3.at[par], g_hbm.at[pl.ds(bb, CB)], sem_w.at[par])
            return carry

        lax.fori_loop(0, NITER, it, 0)

        for j in (NITER - 2, NITER - 1):
            par = j % 2
            pltpu.make_async_copy(
                v3.at[par], g_hbm.at[pl.ds(b_base + j * CB, CB)], sem_w.at[par]
            ).wait()

    @pl.when(c == 0)
    def _():
        run(idxp1, g1)

    @pl.when(c == 1)
    def _():
        run(idxp2, g2)


_gather = pl.kernel(
    _gather_body,
    out_type=[
        jax.ShapeDtypeStruct((B, L, H), jnp.float32),
        jax.ShapeDtypeStruct((B, L, H), jnp.float32),
    ],
    mesh=plsc.VectorSubcoreMesh(core_axis_name="c", subcore_axis_name="s"),
    scratch_types=[
        pltpu.VMEM((NB_W, 128), jnp.int32),
        pltpu.VMEM((2, CB, L, H), jnp.float32),
        pltpu.SemaphoreType.DMA,
        pltpu.SemaphoreType.DMA((2,)),
    ],
    compiler_params=pltpu.CompilerParams(use_tc_tiling_on_sc=True),
)


def kernel(sent1, sent2, emb_table, W, b):
    idxp1 = jnp.pad(sent1.astype(jnp.int32), ((0, 0), (0, 128 - L)))
    idxp2 = jnp.pad(sent2.astype(jnp.int32), ((0, 0), (0, 128 - L)))
    p = _project(emb_table.reshape(V // 8, 8, D), W, b.reshape(1, H))
    o1, o2 = _gather(idxp1, idxp2, p)
    return (o1, o2)


# speedup vs baseline: 1.9249x; 1.0007x over previous
import jax
import jax.numpy as jnp
from jax import lax
from jax.experimental import pallas as pl
from jax.experimental.pallas import tpu as pltpu
from jax.experimental.pallas import tpu_sc as plsc

V = 1000000
D = 64
H = 128
B = 4096
L = 50

NC, NS = 2, 16
CB = 4
NB_W = B // NS
NITER = NB_W // CB


BM3 = 1000
BMP = BM3 * 8


def _proj_body(t_ref, w_ref, b_ref, p_ref):
    t = t_ref[...].reshape(BMP, D)
    p_ref[...] = lax.dot_general(
        t.astype(jnp.bfloat16), w_ref[...].astype(jnp.bfloat16),
        dimension_numbers=(((1,), (1,)), ((), ())),
        preferred_element_type=jnp.float32,
    ) + b_ref[...]


_project = pl.pallas_call(
    _proj_body,
    grid=(V // BMP,),
    in_specs=[
        pl.BlockSpec((BM3, 8, D), lambda i: (i, 0, 0)),
        pl.BlockSpec((H, D), lambda i: (0, 0)),
        pl.BlockSpec((1, H), lambda i: (0, 0)),
    ],
    out_specs=pl.BlockSpec((BMP, H), lambda i: (i, 0)),
    out_shape=jax.ShapeDtypeStruct((V, H), jnp.float32),
)


def _gather_body(idxp1, idxp2, p_hbm, g1, g2, idx_v, v3, sem_g, sem_w):
    c = lax.axis_index("c")
    s = lax.axis_index("s")
    b_base = s * NB_W

    def run(idxp, g_hbm):
        pltpu.sync_copy(idxp.at[pl.ds(b_base, NB_W)], idx_v)

        def it(j, carry):
            par = lax.rem(j, 3)
            bb = b_base + j * CB

            @pl.when(j >= 3)
            def _():
                pltpu.make_async_copy(
                    v3.at[par], g_hbm.at[pl.ds(bb - 3 * CB, CB)], sem_w.at[par]
                ).wait()

            cps = [
                pltpu.async_copy(
                    p_hbm.at[idx_v.at[j * CB + k, pl.ds(0, L)]],
                    v3.at[par, k],
                    sem_g,
                )
                for k in range(CB)
            ]
            for cp in cps:
                cp.wait()
            pltpu.async_copy(v3.at[par], g_hbm.at[pl.ds(bb, CB)], sem_w.at[par])
            return carry

        lax.fori_loop(0, NITER, it, 0)

        for j in (NITER - 3, NITER - 2, NITER - 1):
            par = j % 3
            pltpu.make_async_copy(
                v3.at[par], g_hbm.at[pl.ds(b_base + j * CB, CB)], sem_w.at[par]
            ).wait()

    @pl.when(c == 0)
    def _():
        run(idxp1, g1)

    @pl.when(c == 1)
    def _():
        run(idxp2, g2)


_gather = pl.kernel(
    _gather_body,
    out_type=[
        jax.ShapeDtypeStruct((B, L, H), jnp.float32),
        jax.ShapeDtypeStruct((B, L, H), jnp.float32),
    ],
    mesh=plsc.VectorSubcoreMesh(core_axis_name="c", subcore_axis_name="s"),
    scratch_types=[
        pltpu.VMEM((NB_W, 128), jnp.int32),
        pltpu.VMEM((3, CB, L, H), jnp.float32),
        pltpu.SemaphoreType.DMA,
        pltpu.SemaphoreType.DMA((3,)),
    ],
    compiler_params=pltpu.CompilerParams(use_tc_tiling_on_sc=True),
)


def kernel(sent1, sent2, emb_table, W, b):
    idxp1 = jnp.pad(sent1.astype(jnp.int32), ((0, 0), (0, 128 - L)))
    idxp2 = jnp.pad(sent2.astype(jnp.int32), ((0, 0), (0, 128 - L)))
    p = _project(emb_table.reshape(V // 8, 8, D), W, b.reshape(1, H))
    o1, o2 = _gather(idxp1, idxp2, p)
    return (o1, o2)
